# baseline (speedup 1.0000x reference)
import functools

import jax
import jax.numpy as jnp
from jax import lax
from jax.experimental import pallas as pl
from jax.experimental.pallas import tpu as pltpu
from jax.experimental.pallas import tpu_sc as plsc

VOCAB = 1000000
DIM = 32
B = 1024
L = 200

NW = 32
ROWS = B * L
W_ROWS = ROWS // NW
S_ROWS = 100
N_STREAMS = W_ROWS // S_ROWS
CHUNK = 1600
S_PER_CHUNK = CHUNK // S_ROWS
N_CHUNKS = W_ROWS // CHUNK
REPS = CHUNK // L


def _body(idx_hbm, tok_hbm, pos_hbm, out_hbm, idx_v, pos_v, buf_v, sem):
    wid = lax.axis_index("s") * 2 + lax.axis_index("c")
    base = wid * W_ROWS

    pltpu.sync_copy(idx_hbm.at[wid], idx_v)
    pltpu.sync_copy(pos_hbm.at[pl.ds(0, L)], pos_v)

    for c in range(N_CHUNKS):
        copies = [
            pltpu.async_copy(
                tok_hbm.at[idx_v.at[c * S_PER_CHUNK + j]],
                buf_v.at[pl.ds(j * S_ROWS, S_ROWS)],
                sem,
            )
            for j in range(S_PER_CHUNK)
        ]
        for cp in copies:
            cp.wait()

        def add_pos(l, _):
            p0 = pos_v[l, pl.ds(0, 16)]
            p1 = pos_v[l, pl.ds(16, 16)]
            for q in range(REPS):
                plsc.addupdate(buf_v.at[q * L + l, pl.ds(0, 16)], p0)
                plsc.addupdate(buf_v.at[q * L + l, pl.ds(16, 16)], p1)
            return _

        lax.fori_loop(0, L, add_pos, None)

        pltpu.sync_copy(buf_v, out_hbm.at[pl.ds(base + c * CHUNK, CHUNK)])


@jax.jit
def _lookup(idx3, tok, pos):
    mesh = plsc.VectorSubcoreMesh(core_axis_name="c", subcore_axis_name="s")
    f = functools.partial(
        pl.kernel,
        mesh=mesh,
        out_type=jax.ShapeDtypeStruct((ROWS, DIM), jnp.float32),
        scratch_types=[
            pltpu.VMEM((N_STREAMS, S_ROWS), jnp.int32),
            pltpu.VMEM((L, DIM), jnp.float32),
            pltpu.VMEM((CHUNK, DIM), jnp.float32),
            pltpu.SemaphoreType.DMA,
        ],
        compiler_params=pltpu.CompilerParams(use_tc_tiling_on_sc=False),
    )(_body)
    return f(idx3, tok, pos)


def kernel(indices, token_table, pos_table):
    idx3 = indices.astype(jnp.int32).reshape(NW, N_STREAMS, S_ROWS)
    out = _lookup(idx3, token_table, pos_table)
    return out.reshape(B, L, DIM)


# speedup vs baseline: 1.5913x; 1.5913x over previous
import functools

import jax
import jax.numpy as jnp
from jax import lax
from jax.experimental import pallas as pl
from jax.experimental.pallas import tpu as pltpu
from jax.experimental.pallas import tpu_sc as plsc

VOCAB = 1000000
DIM = 32
B = 1024
L = 200

NW = 32
ROWS = B * L
W_ROWS = ROWS // NW
S_ROWS = 100
N_STREAMS = W_ROWS // S_ROWS
CHUNK = 1600
S_PER_CHUNK = CHUNK // S_ROWS
N_CHUNKS = W_ROWS // CHUNK
REPS = CHUNK // L


def _body(idx_hbm, tok_hbm, pos_hbm, out_hbm, idx_v, pos_v, buf_v, sem):
    wid = lax.axis_index("s") * 2 + lax.axis_index("c")
    base = wid * W_ROWS

    pltpu.sync_copy(idx_hbm.at[wid], idx_v)
    pltpu.sync_copy(pos_hbm, pos_v)

    for c in range(N_CHUNKS):
        copies = [
            pltpu.async_copy(
                tok_hbm.at[idx_v.at[c * S_PER_CHUNK + j]],
                buf_v.at[pl.ds(j * S_ROWS, S_ROWS)],
                sem,
            )
            for j in range(S_PER_CHUNK)
        ]
        for cp in copies:
            cp.wait()

        def add_pos(l, _):
            p0 = pos_v[l, pl.ds(0, 16)]
            p1 = pos_v[l, pl.ds(16, 16)]
            for q in range(REPS):
                plsc.addupdate(buf_v.at[q * L + l, pl.ds(0, 16)], p0)
                plsc.addupdate(buf_v.at[q * L + l, pl.ds(16, 16)], p1)
            return _

        lax.fori_loop(0, L, add_pos, None)

        pltpu.sync_copy(buf_v, out_hbm.at[pl.ds(base + c * CHUNK, CHUNK)])


@jax.jit
def _lookup(idx3, tok, pos):
    mesh = plsc.VectorSubcoreMesh(core_axis_name="c", subcore_axis_name="s")
    f = functools.partial(
        pl.kernel,
        mesh=mesh,
        out_type=jax.ShapeDtypeStruct((ROWS, DIM), jnp.float32),
        scratch_types=[
            pltpu.VMEM((N_STREAMS, S_ROWS), jnp.int32),
            pltpu.VMEM((L, DIM), jnp.float32),
            pltpu.VMEM((CHUNK, DIM), jnp.float32),
            pltpu.SemaphoreType.DMA,
        ],
        compiler_params=pltpu.CompilerParams(use_tc_tiling_on_sc=False),
    )(_body)
    return f(idx3, tok, pos)


def kernel(indices, token_table, pos_table):
    idx3 = indices.astype(jnp.int32).reshape(NW, N_STREAMS, S_ROWS)
    out = _lookup(idx3, token_table, pos_table[:L])
    return out.reshape(B, L, DIM)
